# baseline (speedup 1.0000x reference)
import jax
import jax.numpy as jnp
from jax.experimental import pallas as pl
from jax.experimental.pallas import tpu as pltpu

_B = 8192
_D = 2048
_LD = 256
_NE = 1024
_BB = 256
_NBLK = _B // _BB


def _vqvae_body(x_ref, We1_ref, be1_ref, We2_ref, be2_ref, We3_ref, be3_ref,
                We4_ref, be4_ref, E_ref,
                Wd1_ref, bd1_ref, Wd2_ref, bd2_ref, Wd3_ref, bd3_ref,
                Wd4_ref, bd4_ref,
                pred_ref, qst_ref, enc_ref, loss_ref, perp_ref,
                counts_scr, losssum_scr):
    i = pl.program_id(0)

    @pl.when(i == 0)
    def _init():
        counts_scr[...] = jnp.zeros_like(counts_scr)
        losssum_scr[...] = jnp.zeros_like(losssum_scr)

    x = x_ref[...]
    h = jnp.maximum(jnp.dot(x, We1_ref[...], preferred_element_type=jnp.float32) + be1_ref[...], 0.0)
    h = jnp.maximum(jnp.dot(h, We2_ref[...], preferred_element_type=jnp.float32) + be2_ref[...], 0.0)
    h = jnp.maximum(jnp.dot(h, We3_ref[...], preferred_element_type=jnp.float32) + be3_ref[...], 0.0)
    z = jnp.dot(h, We4_ref[...], preferred_element_type=jnp.float32) + be4_ref[...]

    E = E_ref[...]
    z2 = jnp.sum(z * z, axis=1, keepdims=True)
    e2 = jnp.sum(E * E, axis=1)
    s = jax.lax.dot_general(z, E, (((1,), (1,)), ((), ())),
                            preferred_element_type=jnp.float32)
    dist = z2 + e2[None, :] - 2.0 * s

    min_d = jnp.min(dist, axis=1, keepdims=True)
    iota = jax.lax.broadcasted_iota(jnp.int32, (_BB, _NE), 1)
    idx = jnp.min(jnp.where(dist == min_d, iota, _NE), axis=1, keepdims=True)
    enc = (iota == idx).astype(jnp.float32)

    q = jnp.dot(enc, E, preferred_element_type=jnp.float32)
    qst = z + (q - z)

    enc_ref[...] = enc
    qst_ref[...] = qst

    diff = q - z
    bsum = jnp.sum(jnp.sum(diff * diff, axis=1, keepdims=True), axis=0, keepdims=True)
    losssum_scr[...] += bsum
    counts_scr[...] += jnp.sum(enc, axis=0, keepdims=True)

    @pl.when(i == _NBLK - 1)
    def _fin():
        m = losssum_scr[...] * (1.0 / (_B * _LD))
        loss_ref[...] = m + 0.25 * m
        p = counts_scr[...] * (1.0 / _B)
        ent = jnp.sum(p * jnp.log(p + 1e-10), axis=1, keepdims=True)
        perp_ref[...] = jnp.exp(-ent)

    g = jnp.maximum(jnp.dot(qst, Wd1_ref[...], preferred_element_type=jnp.float32) + bd1_ref[...], 0.0)
    g = jnp.maximum(jnp.dot(g, Wd2_ref[...], preferred_element_type=jnp.float32) + bd2_ref[...], 0.0)
    g = jnp.maximum(jnp.dot(g, Wd3_ref[...], preferred_element_type=jnp.float32) + bd3_ref[...], 0.0)
    pred_ref[...] = jnp.dot(g, Wd4_ref[...], preferred_element_type=jnp.float32) + bd4_ref[...]


def _full(shape):
    return pl.BlockSpec(shape, lambda i: tuple(0 for _ in shape))


def kernel(x, We1, be1, We2, be2, We3, be3, We4, be4, E,
           Wd1, bd1, Wd2, bd2, Wd3, bd3, Wd4, bd4):
    f32 = jnp.float32
    pred, qst, enc, loss11, perp11 = pl.pallas_call(
        _vqvae_body,
        grid=(_NBLK,),
        in_specs=[
            pl.BlockSpec((_BB, _D), lambda i: (i, 0)),
            _full(We1.shape), _full(be1.shape),
            _full(We2.shape), _full(be2.shape),
            _full(We3.shape), _full(be3.shape),
            _full(We4.shape), _full(be4.shape),
            _full(E.shape),
            _full(Wd1.shape), _full(bd1.shape),
            _full(Wd2.shape), _full(bd2.shape),
            _full(Wd3.shape), _full(bd3.shape),
            _full(Wd4.shape), _full(bd4.shape),
        ],
        out_specs=[
            pl.BlockSpec((_BB, _D), lambda i: (i, 0)),
            pl.BlockSpec((_BB, _LD), lambda i: (i, 0)),
            pl.BlockSpec((_BB, _NE), lambda i: (i, 0)),
            pl.BlockSpec((1, 1), lambda i: (0, 0)),
            pl.BlockSpec((1, 1), lambda i: (0, 0)),
        ],
        out_shape=[
            jax.ShapeDtypeStruct((_B, _D), f32),
            jax.ShapeDtypeStruct((_B, _LD), f32),
            jax.ShapeDtypeStruct((_B, _NE), f32),
            jax.ShapeDtypeStruct((1, 1), f32),
            jax.ShapeDtypeStruct((1, 1), f32),
        ],
        scratch_shapes=[
            pltpu.VMEM((1, _NE), f32),
            pltpu.VMEM((1, 1), f32),
        ],
        compiler_params=pltpu.CompilerParams(
            dimension_semantics=("arbitrary",),
        ),
    )(x, We1, be1, We2, be2, We3, be3, We4, be4, E,
      Wd1, bd1, Wd2, bd2, Wd3, bd3, Wd4, bd4)

    return (pred, loss11[0, 0], qst, perp11[0, 0], enc)


# speedup vs baseline: 1.0818x; 1.0818x over previous
import jax
import jax.numpy as jnp
from jax.experimental import pallas as pl
from jax.experimental.pallas import tpu as pltpu

_B = 8192
_D = 2048
_LD = 256
_NE = 1024
_BB = 512
_NBLK = _B // _BB
_BBD = 1024
_NBLKD = _B // _BBD


def _encvq_body(x_ref, We1_ref, be1_ref, We2_ref, be2_ref, We3_ref, be3_ref,
                We4_ref, be4_ref, E_ref,
                qst_ref, enc_ref, loss_ref, perp_ref,
                counts_scr, losssum_scr):
    i = pl.program_id(0)

    @pl.when(i == 0)
    def _init():
        counts_scr[...] = jnp.zeros_like(counts_scr)
        losssum_scr[...] = jnp.zeros_like(losssum_scr)

    x = x_ref[...]
    h = jnp.maximum(jnp.dot(x, We1_ref[...], preferred_element_type=jnp.float32) + be1_ref[...], 0.0)
    h = jnp.maximum(jnp.dot(h, We2_ref[...], preferred_element_type=jnp.float32) + be2_ref[...], 0.0)
    h = jnp.maximum(jnp.dot(h, We3_ref[...], preferred_element_type=jnp.float32) + be3_ref[...], 0.0)
    z = jnp.dot(h, We4_ref[...], preferred_element_type=jnp.float32) + be4_ref[...]

    E = E_ref[...]
    z2 = jnp.sum(z * z, axis=1, keepdims=True)
    e2 = jnp.sum(E * E, axis=1)
    s = jax.lax.dot_general(z, E, (((1,), (1,)), ((), ())),
                            preferred_element_type=jnp.float32)
    dist = z2 + e2[None, :] - 2.0 * s

    min_d = jnp.min(dist, axis=1, keepdims=True)
    iota = jax.lax.broadcasted_iota(jnp.int32, (_BB, _NE), 1)
    idx = jnp.min(jnp.where(dist == min_d, iota, _NE), axis=1, keepdims=True)
    enc = (iota == idx).astype(jnp.float32)

    q = jnp.dot(enc, E, preferred_element_type=jnp.float32)
    qst = z + (q - z)

    enc_ref[...] = enc
    qst_ref[...] = qst

    diff = q - z
    bsum = jnp.sum(jnp.sum(diff * diff, axis=1, keepdims=True), axis=0, keepdims=True)
    losssum_scr[...] += bsum
    counts_scr[...] += jnp.sum(enc, axis=0, keepdims=True)

    @pl.when(i == _NBLK - 1)
    def _fin():
        m = losssum_scr[...] * (1.0 / (_B * _LD))
        loss_ref[...] = m + 0.25 * m
        p = counts_scr[...] * (1.0 / _B)
        ent = jnp.sum(p * jnp.log(p + 1e-10), axis=1, keepdims=True)
        perp_ref[...] = jnp.exp(-ent)


def _dec_body(q_ref, Wd1_ref, bd1_ref, Wd2_ref, bd2_ref, Wd3_ref, bd3_ref,
              Wd4_ref, bd4_ref, out_ref):
    g = jnp.maximum(jnp.dot(q_ref[...], Wd1_ref[...], preferred_element_type=jnp.float32) + bd1_ref[...], 0.0)
    g = jnp.maximum(jnp.dot(g, Wd2_ref[...], preferred_element_type=jnp.float32) + bd2_ref[...], 0.0)
    g = jnp.maximum(jnp.dot(g, Wd3_ref[...], preferred_element_type=jnp.float32) + bd3_ref[...], 0.0)
    out_ref[...] = jnp.dot(g, Wd4_ref[...], preferred_element_type=jnp.float32) + bd4_ref[...]


def _full(shape):
    return pl.BlockSpec(shape, lambda i: tuple(0 for _ in shape))


def kernel(x, We1, be1, We2, be2, We3, be3, We4, be4, E,
           Wd1, bd1, Wd2, bd2, Wd3, bd3, Wd4, bd4):
    f32 = jnp.float32
    qst, enc, loss11, perp11 = pl.pallas_call(
        _encvq_body,
        grid=(_NBLK,),
        in_specs=[
            pl.BlockSpec((_BB, _D), lambda i: (i, 0)),
            _full(We1.shape), _full(be1.shape),
            _full(We2.shape), _full(be2.shape),
            _full(We3.shape), _full(be3.shape),
            _full(We4.shape), _full(be4.shape),
            _full(E.shape),
        ],
        out_specs=[
            pl.BlockSpec((_BB, _LD), lambda i: (i, 0)),
            pl.BlockSpec((_BB, _NE), lambda i: (i, 0)),
            pl.BlockSpec((1, 1), lambda i: (0, 0)),
            pl.BlockSpec((1, 1), lambda i: (0, 0)),
        ],
        out_shape=[
            jax.ShapeDtypeStruct((_B, _LD), f32),
            jax.ShapeDtypeStruct((_B, _NE), f32),
            jax.ShapeDtypeStruct((1, 1), f32),
            jax.ShapeDtypeStruct((1, 1), f32),
        ],
        scratch_shapes=[
            pltpu.VMEM((1, _NE), f32),
            pltpu.VMEM((1, 1), f32),
        ],
        compiler_params=pltpu.CompilerParams(
            dimension_semantics=("arbitrary",),
        ),
    )(x, We1, be1, We2, be2, We3, be3, We4, be4, E)

    pred = pl.pallas_call(
        _dec_body,
        grid=(_NBLKD,),
        in_specs=[
            pl.BlockSpec((_BBD, _LD), lambda i: (i, 0)),
            _full(Wd1.shape), _full(bd1.shape),
            _full(Wd2.shape), _full(bd2.shape),
            _full(Wd3.shape), _full(bd3.shape),
            _full(Wd4.shape), _full(bd4.shape),
        ],
        out_specs=pl.BlockSpec((_BBD, _D), lambda i: (i, 0)),
        out_shape=jax.ShapeDtypeStruct((_B, _D), f32),
        compiler_params=pltpu.CompilerParams(
            dimension_semantics=("parallel",),
        ),
    )(qst, Wd1, bd1, Wd2, bd2, Wd3, bd3, Wd4, bd4)

    return (pred, loss11[0, 0], qst, perp11[0, 0], enc)
